# baseline (speedup 1.0000x reference)
import jax
import jax.numpy as jnp
from jax.experimental import pallas as pl
from jax.experimental.pallas import tpu as pltpu

N = 4096
F = 256
H = 128
TAU = 0.5
HC = N // 2
BM = 256
NI = N // BM
NB = 5
LOG2E = 1.4426950408889634


def _cl_gcn_kernel(x1_ref, x2_ref, adj1_ref, adj2_ref, clm_ref,
                   w11_ref, b11_ref, w12_ref, b12_ref,
                   w21_ref, b21_ref, w22_ref, b22_ref,
                   z1_ref, z2_ref, loss_ref,
                   adj_scr, sup1_scr, sup2_scr, s2a_scr, s2b_scr,
                   buf0, buf1, buf2, buf3, buf4, sems,
                   zstage0, zstage1, zsems, acc_ref):
    bufs = (buf0, buf1, buf2, buf3, buf4)
    zstages = (zstage0, zstage1)

    def x_task(x_hbm, k):
        def start(b, sem):
            pltpu.make_async_copy(
                x_hbm.at[pl.ds(k * BM, BM), :], b.at[:, :F], sem).start()

        def wait(b, sem):
            pltpu.make_async_copy(
                x_hbm.at[pl.ds(k * BM, BM), :], b.at[:, :F], sem).wait()
        return start, wait

    def row_task(a_hbm, k):
        def start(b, sem):
            pltpu.make_async_copy(
                a_hbm.at[pl.ds(k * BM, BM), :], b, sem).start()

        def wait(b, sem):
            pltpu.make_async_copy(
                a_hbm.at[pl.ds(k * BM, BM), :], b, sem).wait()
        return start, wait

    tasks = ([x_task(x1_ref, k) for k in range(NI)] +
             [x_task(x2_ref, k) for k in range(NI)] +
             [row_task(adj1_ref, k) for k in range(NI)] +
             [row_task(adj2_ref, k) for k in range(NI)] +
             [row_task(clm_ref, k) for k in range(NI)])
    T = len(tasks)

    def start_task(t):
        if t < T:
            tasks[t][0](bufs[t % NB], sems.at[t % NB])

    def wait_task(t):
        tasks[t][1](bufs[t % NB], sems.at[t % NB])

    for t in range(NB):
        start_task(t)

    t = 0
    for tower in range(2):
        sup_scr = sup1_scr if tower == 0 else sup2_scr
        w_ref = w11_ref if tower == 0 else w21_ref
        for k in range(NI):
            wait_task(t)
            xb = bufs[t % NB][:, :F].astype(jnp.bfloat16)
            sup = jnp.dot(xb, w_ref[...], preferred_element_type=jnp.float32)
            sup_scr[pl.ds(k * BM, BM), :] = sup.astype(jnp.bfloat16)
            start_task(t + NB)
            t += 1

    for k in range(NI):
        wait_task(t)
        ab = bufs[t % NB][...].astype(jnp.bfloat16)
        adj_scr[pl.ds(k * BM, BM), :] = ab
        acc = jnp.dot(ab, sup1_scr[...], preferred_element_type=jnp.float32)
        h = jnp.maximum(acc + b11_ref[...], 0.0)
        s2 = jnp.dot(h.astype(jnp.bfloat16), w12_ref[...],
                     preferred_element_type=jnp.float32)
        s2a_scr[pl.ds(k * BM, BM), :] = s2.astype(jnp.bfloat16)
        start_task(t + NB)
        t += 1

    for k in range(NI):
        a1 = adj_scr[pl.ds(k * BM, BM), :]
        z1 = jnp.dot(a1, s2a_scr[...],
                     preferred_element_type=jnp.float32) + b12_ref[...]
        if k >= 2:
            pltpu.make_async_copy(zstages[k % 2],
                                  z1_ref.at[pl.ds((k - 2) * BM, BM), :],
                                  zsems.at[k % 2]).wait()
        zstages[k % 2][...] = z1
        pltpu.make_async_copy(zstages[k % 2],
                              z1_ref.at[pl.ds(k * BM, BM), :],
                              zsems.at[k % 2]).start()
        r1 = jax.lax.rsqrt(jnp.sum(z1 * z1, axis=1, keepdims=True))
        sup1_scr[pl.ds(k * BM, BM), :H] = (z1 * r1 * (LOG2E / TAU)
                                           ).astype(jnp.bfloat16)
        wait_task(t)
        ab = bufs[t % NB][...].astype(jnp.bfloat16)
        adj_scr[pl.ds(k * BM, BM), :] = ab
        acc = jnp.dot(ab, sup2_scr[...], preferred_element_type=jnp.float32)
        h = jnp.maximum(acc + b21_ref[...], 0.0)
        s2 = jnp.dot(h.astype(jnp.bfloat16), w22_ref[...],
                     preferred_element_type=jnp.float32)
        s2b_scr[pl.ds(k * BM, BM), :] = s2.astype(jnp.bfloat16)
        start_task(t + NB)
        t += 1

    for k in (NI, NI + 1):
        pltpu.make_async_copy(zstages[k % 2],
                              z1_ref.at[pl.ds((k - 2) * BM, BM), :],
                              zsems.at[k % 2]).wait()

    for k in range(NI):
        a2 = adj_scr[pl.ds(k * BM, BM), :]
        z2 = jnp.dot(a2, s2b_scr[...],
                     preferred_element_type=jnp.float32) + b22_ref[...]
        if k >= 2:
            pltpu.make_async_copy(zstages[k % 2],
                                  z2_ref.at[pl.ds((k - 2) * BM, BM), :],
                                  zsems.at[k % 2]).wait()
        zstages[k % 2][...] = z2
        pltpu.make_async_copy(zstages[k % 2],
                              z2_ref.at[pl.ds(k * BM, BM), :],
                              zsems.at[k % 2]).start()
        r2 = jax.lax.rsqrt(jnp.sum(z2 * z2, axis=1, keepdims=True))
        sup2_scr[pl.ds(k * BM, BM), :H] = (z2 * r2).astype(jnp.bfloat16)

    for k in (NI, NI + 1):
        pltpu.make_async_copy(zstages[k % 2],
                              z2_ref.at[pl.ds((k - 2) * BM, BM), :],
                              zsems.at[k % 2]).wait()

    for k in range(NI):
        wait_task(t)
        clm = bufs[t % NB]
        z1s = sup1_scr[pl.ds(k * BM, BM), :H]
        rs = jnp.zeros((BM, 1), dtype=jnp.float32)
        ws = jnp.zeros((BM, 1), dtype=jnp.float32)
        for half in range(2):
            z2s = sup2_scr[pl.ds(half * HC, HC), :H]
            s = jax.lax.dot_general(z1s, z2s, (((1,), (1,)), ((), ())),
                                    preferred_element_type=jnp.float32)
            pexp = jnp.exp2(s)
            rs = rs + jnp.sum(pexp, axis=1, keepdims=True)
            ws = ws + jnp.sum(pexp * clm[:, half * HC:(half + 1) * HC],
                              axis=1, keepdims=True)
        part = jnp.sum(jnp.log(rs + 1e-8) - jnp.log(ws))
        if k == 0:
            acc_ref[0] = part
        else:
            acc_ref[0] += part
        start_task(t + NB)
        t += 1

    loss_ref[...] = jnp.full((1, 1), acc_ref[0] * (1.0 / N),
                             dtype=jnp.float32)


def _cl_gcn(x1, adj1, x2, adj2, clm,
            W11, b11, W12, b12, W21, b21, W22, b22):
    z1, z2, loss = pl.pallas_call(
        _cl_gcn_kernel,
        in_specs=[
            pl.BlockSpec(memory_space=pl.ANY),
            pl.BlockSpec(memory_space=pl.ANY),
            pl.BlockSpec(memory_space=pl.ANY),
            pl.BlockSpec(memory_space=pl.ANY),
            pl.BlockSpec(memory_space=pl.ANY),
            pl.BlockSpec((F, F), lambda: (0, 0)),
            pl.BlockSpec((1, F), lambda: (0, 0)),
            pl.BlockSpec((F, H), lambda: (0, 0)),
            pl.BlockSpec((1, H), lambda: (0, 0)),
            pl.BlockSpec((F, F), lambda: (0, 0)),
            pl.BlockSpec((1, F), lambda: (0, 0)),
            pl.BlockSpec((F, H), lambda: (0, 0)),
            pl.BlockSpec((1, H), lambda: (0, 0)),
        ],
        out_specs=[
            pl.BlockSpec(memory_space=pl.ANY),
            pl.BlockSpec(memory_space=pl.ANY),
            pl.BlockSpec((1, 1), lambda: (0, 0)),
        ],
        out_shape=[
            jax.ShapeDtypeStruct((N, H), jnp.float32),
            jax.ShapeDtypeStruct((N, H), jnp.float32),
            jax.ShapeDtypeStruct((1, 1), jnp.float32),
        ],
        scratch_shapes=[
            pltpu.VMEM((N, N), jnp.bfloat16),
            pltpu.VMEM((N, F), jnp.bfloat16),
            pltpu.VMEM((N, F), jnp.bfloat16),
            pltpu.VMEM((N, H), jnp.bfloat16),
            pltpu.VMEM((N, H), jnp.bfloat16),
            pltpu.VMEM((BM, N), jnp.float32),
            pltpu.VMEM((BM, N), jnp.float32),
            pltpu.VMEM((BM, N), jnp.float32),
            pltpu.VMEM((BM, N), jnp.float32),
            pltpu.VMEM((BM, N), jnp.float32),
            pltpu.SemaphoreType.DMA((NB,)),
            pltpu.VMEM((BM, H), jnp.float32),
            pltpu.VMEM((BM, H), jnp.float32),
            pltpu.SemaphoreType.DMA((2,)),
            pltpu.SMEM((1,), jnp.float32),
        ],
        compiler_params=pltpu.CompilerParams(
            vmem_limit_bytes=63 * 1024 * 1024,
        ),
    )(x1, x2, adj1, adj2, clm,
      W11.astype(jnp.bfloat16), b11.reshape(1, F),
      W12.astype(jnp.bfloat16), b12.reshape(1, H),
      W21.astype(jnp.bfloat16), b21.reshape(1, F),
      W22.astype(jnp.bfloat16), b22.reshape(1, H))
    return z1, z2, loss.reshape(())


def kernel(x1, adj1, x2, adj2, clm, W11, b11, W12, b12, W21, b21, W22, b22):
    z1, z2, loss = _cl_gcn(x1, adj1, x2, adj2, clm,
                           W11, b11, W12, b12, W21, b21, W22, b22)
    return (z1, z2, loss)


# speedup vs baseline: 1.0308x; 1.0308x over previous
import jax
import jax.numpy as jnp
from jax.experimental import pallas as pl
from jax.experimental.pallas import tpu as pltpu

N = 4096
F = 256
H = 128
TAU = 0.5
HC = N // 2
BM = 256
NI = N // BM
NB = 5
LOG2E = 1.4426950408889634


def _cl_gcn_kernel(x1_ref, x2_ref, adj1_ref, adj2_ref, clm_ref,
                   w11_ref, b11_ref, w12_ref, b12_ref,
                   w21_ref, b21_ref, w22_ref, b22_ref,
                   z1_ref, z2_ref, loss_ref,
                   adj_scr, sup1_scr, sup2_scr, s2a_scr, s2b_scr,
                   buf0, buf1, buf2, buf3, buf4, sems, acc_ref):
    bufs = (buf0, buf1, buf2, buf3, buf4)

    def x_task(x_hbm, k):
        def start(b, sem):
            pltpu.make_async_copy(
                x_hbm.at[pl.ds(k * BM, BM), :], b.at[:, :F], sem).start()

        def wait(b, sem):
            pltpu.make_async_copy(
                x_hbm.at[pl.ds(k * BM, BM), :], b.at[:, :F], sem).wait()
        return start, wait

    def row_task(a_hbm, k):
        def start(b, sem):
            pltpu.make_async_copy(
                a_hbm.at[pl.ds(k * BM, BM), :], b, sem).start()

        def wait(b, sem):
            pltpu.make_async_copy(
                a_hbm.at[pl.ds(k * BM, BM), :], b, sem).wait()
        return start, wait

    tasks = ([x_task(x1_ref, k) for k in range(NI)] +
             [x_task(x2_ref, k) for k in range(NI)] +
             [row_task(adj1_ref, k) for k in range(NI)] +
             [row_task(adj2_ref, k) for k in range(NI)] +
             [row_task(clm_ref, k) for k in range(NI)])
    T = len(tasks)

    def start_task(t):
        if t < T:
            tasks[t][0](bufs[t % NB], sems.at[t % NB])

    def wait_task(t):
        tasks[t][1](bufs[t % NB], sems.at[t % NB])

    for t in range(NB):
        start_task(t)

    t = 0
    for tower in range(2):
        sup_scr = sup1_scr if tower == 0 else sup2_scr
        w_ref = w11_ref if tower == 0 else w21_ref
        for k in range(NI):
            wait_task(t)
            xb = bufs[t % NB][:, :F].astype(jnp.bfloat16)
            sup = jnp.dot(xb, w_ref[...], preferred_element_type=jnp.float32)
            sup_scr[pl.ds(k * BM, BM), :] = sup.astype(jnp.bfloat16)
            start_task(t + NB)
            t += 1

    for k in range(NI):
        wait_task(t)
        ab = bufs[t % NB][...].astype(jnp.bfloat16)
        adj_scr[pl.ds(k * BM, BM), :] = ab
        acc = jnp.dot(ab, sup1_scr[...], preferred_element_type=jnp.float32)
        h = jnp.maximum(acc + b11_ref[...], 0.0)
        s2 = jnp.dot(h.astype(jnp.bfloat16), w12_ref[...],
                     preferred_element_type=jnp.float32)
        s2a_scr[pl.ds(k * BM, BM), :] = s2.astype(jnp.bfloat16)
        start_task(t + NB)
        t += 1

    for k in range(NI):
        a1 = adj_scr[pl.ds(k * BM, BM), :]
        z1 = jnp.dot(a1, s2a_scr[...],
                     preferred_element_type=jnp.float32) + b12_ref[...]
        z1_ref[pl.ds(k * BM, BM), :] = z1
        r1 = jax.lax.rsqrt(jnp.sum(z1 * z1, axis=1, keepdims=True))
        sup1_scr[pl.ds(k * BM, BM), :H] = (z1 * r1 * (LOG2E / TAU)
                                           ).astype(jnp.bfloat16)
        wait_task(t)
        ab = bufs[t % NB][...].astype(jnp.bfloat16)
        adj_scr[pl.ds(k * BM, BM), :] = ab
        acc = jnp.dot(ab, sup2_scr[...], preferred_element_type=jnp.float32)
        h = jnp.maximum(acc + b21_ref[...], 0.0)
        s2 = jnp.dot(h.astype(jnp.bfloat16), w22_ref[...],
                     preferred_element_type=jnp.float32)
        s2b_scr[pl.ds(k * BM, BM), :] = s2.astype(jnp.bfloat16)
        start_task(t + NB)
        t += 1

    for k in range(NI):
        a2 = adj_scr[pl.ds(k * BM, BM), :]
        z2 = jnp.dot(a2, s2b_scr[...],
                     preferred_element_type=jnp.float32) + b22_ref[...]
        z2_ref[pl.ds(k * BM, BM), :] = z2
        r2 = jax.lax.rsqrt(jnp.sum(z2 * z2, axis=1, keepdims=True))
        sup2_scr[pl.ds(k * BM, BM), :H] = (z2 * r2).astype(jnp.bfloat16)

    for k in range(NI):
        wait_task(t)
        clm = bufs[t % NB]
        z1s = sup1_scr[pl.ds(k * BM, BM), :H]
        rs = jnp.zeros((BM, 1), dtype=jnp.float32)
        ws = jnp.zeros((BM, 1), dtype=jnp.float32)
        for half in range(2):
            z2s = sup2_scr[pl.ds(half * HC, HC), :H]
            s = jax.lax.dot_general(z1s, z2s, (((1,), (1,)), ((), ())),
                                    preferred_element_type=jnp.float32)
            pexp = jnp.exp2(s)
            rs = rs + jnp.sum(pexp, axis=1, keepdims=True)
            ws = ws + jnp.sum(pexp * clm[:, half * HC:(half + 1) * HC],
                              axis=1, keepdims=True)
        part = jnp.sum(jnp.log(rs + 1e-8) - jnp.log(ws))
        if k == 0:
            acc_ref[0] = part
        else:
            acc_ref[0] += part
        start_task(t + NB)
        t += 1

    loss_ref[...] = jnp.full((1, 1), acc_ref[0] * (1.0 / N),
                             dtype=jnp.float32)


def _cl_gcn(x1, adj1, x2, adj2, clm,
            W11, b11, W12, b12, W21, b21, W22, b22):
    z1, z2, loss = pl.pallas_call(
        _cl_gcn_kernel,
        in_specs=[
            pl.BlockSpec(memory_space=pl.ANY),
            pl.BlockSpec(memory_space=pl.ANY),
            pl.BlockSpec(memory_space=pl.ANY),
            pl.BlockSpec(memory_space=pl.ANY),
            pl.BlockSpec(memory_space=pl.ANY),
            pl.BlockSpec((F, F), lambda: (0, 0)),
            pl.BlockSpec((1, F), lambda: (0, 0)),
            pl.BlockSpec((F, H), lambda: (0, 0)),
            pl.BlockSpec((1, H), lambda: (0, 0)),
            pl.BlockSpec((F, F), lambda: (0, 0)),
            pl.BlockSpec((1, F), lambda: (0, 0)),
            pl.BlockSpec((F, H), lambda: (0, 0)),
            pl.BlockSpec((1, H), lambda: (0, 0)),
        ],
        out_specs=[
            pl.BlockSpec((N, H), lambda: (0, 0)),
            pl.BlockSpec((N, H), lambda: (0, 0)),
            pl.BlockSpec((1, 1), lambda: (0, 0)),
        ],
        out_shape=[
            jax.ShapeDtypeStruct((N, H), jnp.float32),
            jax.ShapeDtypeStruct((N, H), jnp.float32),
            jax.ShapeDtypeStruct((1, 1), jnp.float32),
        ],
        scratch_shapes=[
            pltpu.VMEM((N, N), jnp.bfloat16),
            pltpu.VMEM((N, F), jnp.bfloat16),
            pltpu.VMEM((N, F), jnp.bfloat16),
            pltpu.VMEM((N, H), jnp.bfloat16),
            pltpu.VMEM((N, H), jnp.bfloat16),
            pltpu.VMEM((BM, N), jnp.float32),
            pltpu.VMEM((BM, N), jnp.float32),
            pltpu.VMEM((BM, N), jnp.float32),
            pltpu.VMEM((BM, N), jnp.float32),
            pltpu.VMEM((BM, N), jnp.float32),
            pltpu.SemaphoreType.DMA((NB,)),
            pltpu.SMEM((1,), jnp.float32),
        ],
        compiler_params=pltpu.CompilerParams(
            vmem_limit_bytes=63 * 1024 * 1024,
        ),
    )(x1, x2, adj1, adj2, clm,
      W11.astype(jnp.bfloat16), b11.reshape(1, F),
      W12.astype(jnp.bfloat16), b12.reshape(1, H),
      W21.astype(jnp.bfloat16), b21.reshape(1, F),
      W22.astype(jnp.bfloat16), b22.reshape(1, H))
    return z1, z2, loss.reshape(())


def kernel(x1, adj1, x2, adj2, clm, W11, b11, W12, b12, W21, b21, W22, b22):
    z1, z2, loss = _cl_gcn(x1, adj1, x2, adj2, clm,
                           W11, b11, W12, b12, W21, b21, W22, b22)
    return (z1, z2, loss)
